# Initial kernel scaffold; baseline (speedup 1.0000x reference)
import jax
import jax.numpy as jnp
from jax.experimental import pallas as pl


def kernel(token_ids, token_table, pos_table):
    raise NotImplementedError("write your pallas kernel here")


# speedup vs baseline: 1.5117x; 1.5117x over previous
import functools

import jax
import jax.numpy as jnp
from jax import lax
from jax.experimental import pallas as pl
from jax.experimental.pallas import tpu as pltpu
from jax.experimental.pallas import tpu_sc as plsc

NUM_TOKENS = 100000
MAX_LENGTH = 2048
EMB_SIZE = 768
BATCH = 4
SEQ_LEN = 2048

L = 16
NW = 32
S_PER_W = SEQ_LEN // NW
CHUNKS = EMB_SIZE // L


def _tok_pos_kernel(ids_hbm, table_hbm, pos_hbm, out_hbm, idx_v, pos_v, rows_v, sem):
    wid = lax.axis_index("s") * 2 + lax.axis_index("c")
    base = wid * S_PER_W

    pltpu.sync_copy(pos_hbm.at[pl.ds(base, S_PER_W)], pos_v)

    for b in range(BATCH):
        pltpu.sync_copy(ids_hbm.at[b, pl.ds(base, S_PER_W)], idx_v)
        pltpu.async_copy(table_hbm.at[idx_v], rows_v, sem).wait()

        def add_row(r, _):
            for j in range(CHUNKS):
                sl = pl.ds(j * L, L)
                rows_v[r, sl] = rows_v[r, sl] + pos_v[r, sl]
            return _

        lax.fori_loop(0, S_PER_W, add_row, None)

        pltpu.sync_copy(rows_v, out_hbm.at[b, pl.ds(base, S_PER_W), :])


@jax.jit
def _tok_pos(token_ids, token_table, pos_table):
    mesh = plsc.VectorSubcoreMesh(core_axis_name="c", subcore_axis_name="s")
    run = functools.partial(
        pl.kernel,
        mesh=mesh,
        out_type=jax.ShapeDtypeStruct((BATCH, SEQ_LEN, EMB_SIZE), jnp.float32),
        scratch_types=[
            pltpu.VMEM((S_PER_W,), jnp.int32),
            pltpu.VMEM((S_PER_W, EMB_SIZE), jnp.float32),
            pltpu.VMEM((S_PER_W, EMB_SIZE), jnp.float32),
            pltpu.SemaphoreType.DMA,
        ],
    )(_tok_pos_kernel)
    return run(token_ids, token_table, pos_table)


def kernel(token_ids, token_table, pos_table):
    return _tok_pos(token_ids.astype(jnp.int32), token_table, pos_table)
